# Initial kernel scaffold; baseline (speedup 1.0000x reference)
import jax
import jax.numpy as jnp
from jax.experimental import pallas as pl


def kernel(x, edge_index, batch, Win, bin_, W1, b1, g1, be1, rm1, rv1, W2, b2, g2, be2, rm2, rv2, W3, b3, g3, be3, rm3, rv3, Wo1, bo1, Wo2, bo2):
    raise NotImplementedError("write your pallas kernel here")


# speedup vs baseline: 1.6216x; 1.6216x over previous
import jax
import jax.numpy as jnp
from jax import lax
from jax.experimental import pallas as pl
from jax.experimental.pallas import tpu as pltpu
from jax.experimental.pallas import tpu_sc as plsc

_N = 50000
_E = 800000
_DIN = 128
_DH = 64
_DE = 32
_BG = 64

_NC, _NS = 2, 16
_K = 128
_NBL = 392
_NBD = 196
_EP = _NS * _NBL * _K
_JR = _N
_RA = 30336
_RS = 30328
_NQ = 2
_NOUT = _NQ * _RS
_RPT = _RA // _NS
_HALF = _DH // 2

_mesh = plsc.VectorSubcoreMesh(core_axis_name="c", subcore_axis_name="s",
                               num_cores=_NC, num_subcores=_NS)


def _fill2(ref, nrows, value):
    v = jnp.full((16,), value, jnp.float32)

    def body(r, carry):
        ref[r, pl.ds(0, 16)] = v
        ref[r, pl.ds(16, 16)] = v
        return carry

    lax.fori_loop(0, nrows, body, 0)


def _remap(idx_ref, base):
    def body(k, carry):
        v = idx_ref[pl.ds(k * 16, 16)]
        lo = v - base
        ok = (lo >= 0) & (lo < _RS)
        idx_ref[pl.ds(k * 16, 16)] = jnp.where(ok, lo, _RS)
        return carry

    lax.fori_loop(0, _K // 16, body, 0)


def _dump(acc, out, c, s, qbase):
    @pl.when(s < _NS - 1)
    def _():
        pltpu.sync_copy(acc.at[pl.ds(s * _RPT, _RPT)],
                        out.at[c, pl.ds(qbase + s * _RPT, _RPT)])

    @pl.when(s == _NS - 1)
    def _():
        pltpu.sync_copy(acc.at[pl.ds((_NS - 1) * _RPT, _RPT - 8)],
                        out.at[c, pl.ds(qbase + (_NS - 1) * _RPT, _RPT - 8)])


def _seg_body(table, src2, dst2, out, isa_, isb_, ida_, idb_, rowa, rowb, zb,
              acc, s_is_a, s_is_b, s_id_a, s_id_b, s_g_a, s_g_b):
    c = lax.axis_index("c")
    s = lax.axis_index("s")
    _fill2(zb, _RPT, 0.0)

    def istart(j, islot, dslot, si, sd):
        pltpu.make_async_copy(src2.at[c, s, j], islot, si).start()
        pltpu.make_async_copy(dst2.at[s, j], dslot, sd).start()

    def iwait_src(j, islot, si):
        pltpu.make_async_copy(src2.at[c, s, j], islot, si).wait()

    def iwait_dst(j, dslot, sd):
        pltpu.make_async_copy(dst2.at[s, j], dslot, sd).wait()

    def gstart(islot, rslot, sg):
        pltpu.make_async_copy(table.at[islot], rslot, sg).start()

    def gwait(islot, rslot, sg):
        pltpu.make_async_copy(table.at[islot], rslot, sg).wait()

    for q in range(_NQ):
        base = q * _RS
        pltpu.sync_copy(zb, acc.at[pl.ds(s * _RPT, _RPT)])
        plsc.subcore_barrier()

        istart(0, isa_, ida_, s_is_a, s_id_a)
        iwait_src(0, isa_, s_is_a)
        gstart(isa_, rowa, s_g_a)

        def body(jo, carry):
            j0 = 2 * jo
            istart(j0 + 1, isb_, idb_, s_is_b, s_id_b)
            gwait(isa_, rowa, s_g_a)
            iwait_src(j0 + 1, isb_, s_is_b)
            gstart(isb_, rowb, s_g_b)
            iwait_dst(j0, ida_, s_id_a)
            _remap(ida_, base)
            pltpu.sync_copy(rowa, acc.at[ida_], add=True)
            gwait(isb_, rowb, s_g_b)

            @pl.when(jo < _NBL // 2 - 1)
            def _():
                istart(j0 + 2, isa_, ida_, s_is_a, s_id_a)
                iwait_src(j0 + 2, isa_, s_is_a)
                gstart(isa_, rowa, s_g_a)

            iwait_dst(j0 + 1, idb_, s_id_b)
            _remap(idb_, base)
            pltpu.sync_copy(rowb, acc.at[idb_], add=True)
            return carry

        lax.fori_loop(0, _NBL // 2, body, 0)
        plsc.subcore_barrier()
        _dump(acc, out, c, s, base)


_seg_call = pl.kernel(
    _seg_body,
    out_type=jax.ShapeDtypeStruct((_NC, _NOUT, _HALF), jnp.float32),
    mesh=_mesh,
    compiler_params=pltpu.CompilerParams(use_tc_tiling_on_sc=False),
    scratch_types=[
        pltpu.VMEM((_K,), jnp.int32),
        pltpu.VMEM((_K,), jnp.int32),
        pltpu.VMEM((_K,), jnp.int32),
        pltpu.VMEM((_K,), jnp.int32),
        pltpu.VMEM((_K, _HALF), jnp.float32),
        pltpu.VMEM((_K, _HALF), jnp.float32),
        pltpu.VMEM((_RPT, _HALF), jnp.float32),
        pltpu.VMEM_SHARED((_RA, _HALF), jnp.float32),
        pltpu.SemaphoreType.DMA,
        pltpu.SemaphoreType.DMA,
        pltpu.SemaphoreType.DMA,
        pltpu.SemaphoreType.DMA,
        pltpu.SemaphoreType.DMA,
        pltpu.SemaphoreType.DMA,
    ],
)


_RB = 1000
_GRID = _N // _RB


def _mm_t(a, w):
    return lax.dot_general(a, w, (((1,), (1,)), ((), ())),
                           preferred_element_type=jnp.float32)


def _tc0_body(x_ref, win_ref, bin_ref, w1_ref, dd_ref, hw_ref, dis_ref):
    deg = dd_ref[0, :, 0:1] + 1.0
    dis = lax.rsqrt(deg)
    h0 = jnp.maximum(_mm_t(x_ref[...], win_ref[...]) + bin_ref[...], 0.0)
    hw = _mm_t(h0, w1_ref[...]) * dis
    hw_ref[0] = hw[:, :_HALF]
    hw_ref[1] = hw[:, _HALF:]
    dis_ref[...] = dis


_tc0_call = pl.pallas_call(
    _tc0_body,
    grid=(_GRID,),
    in_specs=[
        pl.BlockSpec((_RB, _DIN), lambda i: (i, 0)),
        pl.BlockSpec((_DH, _DIN), lambda i: (0, 0)),
        pl.BlockSpec((1, _DH), lambda i: (0, 0)),
        pl.BlockSpec((_DH, _DH), lambda i: (0, 0)),
        pl.BlockSpec((_NC, _RB, _HALF), lambda i: (0, i, 0)),
    ],
    out_specs=[
        pl.BlockSpec((_NC, _RB, _HALF), lambda i: (0, i, 0)),
        pl.BlockSpec((_RB, 1), lambda i: (i, 0)),
    ],
    out_shape=[
        jax.ShapeDtypeStruct((_NC, _N, _HALF), jnp.float32),
        jax.ShapeDtypeStruct((_N, 1), jnp.float32),
    ],
)


def _bn_relu(sd_ref, hwp_ref, dis_ref, b_ref, g_ref, be_ref, rm_ref, rv_ref):
    sc = g_ref[...] * lax.rsqrt(rv_ref[...] + 1e-5)
    tb = (b_ref[...] - rm_ref[...]) * sc + be_ref[...]
    seg = jnp.concatenate([sd_ref[0] + hwp_ref[0], sd_ref[1] + hwp_ref[1]],
                          axis=1) * dis_ref[...]
    return jnp.maximum(seg * sc + tb, 0.0)


def _tcmid_body(sd_ref, hwp_ref, dis_ref, b_ref, g_ref, be_ref, rm_ref,
                rv_ref, wn_ref, hw_ref):
    h = _bn_relu(sd_ref, hwp_ref, dis_ref, b_ref, g_ref, be_ref, rm_ref, rv_ref)
    hw = _mm_t(h, wn_ref[...]) * dis_ref[...]
    hw_ref[0] = hw[:, :_HALF]
    hw_ref[1] = hw[:, _HALF:]


_tcmid_call = pl.pallas_call(
    _tcmid_body,
    grid=(_GRID,),
    in_specs=[
        pl.BlockSpec((_NC, _RB, _HALF), lambda i: (0, i, 0)),
        pl.BlockSpec((_NC, _RB, _HALF), lambda i: (0, i, 0)),
        pl.BlockSpec((_RB, 1), lambda i: (i, 0)),
        pl.BlockSpec((1, _DH), lambda i: (0, 0)),
        pl.BlockSpec((1, _DH), lambda i: (0, 0)),
        pl.BlockSpec((1, _DH), lambda i: (0, 0)),
        pl.BlockSpec((1, _DH), lambda i: (0, 0)),
        pl.BlockSpec((1, _DH), lambda i: (0, 0)),
        pl.BlockSpec((_DH, _DH), lambda i: (0, 0)),
    ],
    out_specs=pl.BlockSpec((_NC, _RB, _HALF), lambda i: (0, i, 0)),
    out_shape=jax.ShapeDtypeStruct((_NC, _N, _HALF), jnp.float32),
)


def _tc3_body(sd_ref, hwp_ref, dis_ref, b_ref, g_ref, be_ref, rm_ref, rv_ref,
              batch_ref, wo1_ref, bo1_ref, wo2_ref, bo2_ref, out_ref,
              accp, accc):
    i = pl.program_id(0)

    @pl.when(i == 0)
    def _():
        accp[...] = jnp.zeros_like(accp)
        accc[...] = jnp.zeros_like(accc)

    h = _bn_relu(sd_ref, hwp_ref, dis_ref, b_ref, g_ref, be_ref, rm_ref, rv_ref)
    gid = lax.broadcasted_iota(jnp.int32, (_RB, _BG), 1)
    oh = (batch_ref[...] == gid).astype(jnp.float32)
    accp[...] += lax.dot_general(oh, h, (((0,), (0,)), ((), ())),
                                 preferred_element_type=jnp.float32)
    accc[...] += jnp.sum(oh, axis=0, keepdims=True)

    @pl.when(i == _GRID - 1)
    def _():
        cnt = jnp.reshape(jnp.maximum(accc[...], 1.0), (_BG, 1))
        pooled = accp[...] / cnt
        hid = jnp.maximum(_mm_t(pooled, wo1_ref[...]) + bo1_ref[...], 0.0)
        out_ref[...] = _mm_t(hid, wo2_ref[...]) + bo2_ref[...]


_tc3_call = pl.pallas_call(
    _tc3_body,
    grid=(_GRID,),
    in_specs=[
        pl.BlockSpec((_NC, _RB, _HALF), lambda i: (0, i, 0)),
        pl.BlockSpec((_NC, _RB, _HALF), lambda i: (0, i, 0)),
        pl.BlockSpec((_RB, 1), lambda i: (i, 0)),
        pl.BlockSpec((1, _DH), lambda i: (0, 0)),
        pl.BlockSpec((1, _DH), lambda i: (0, 0)),
        pl.BlockSpec((1, _DH), lambda i: (0, 0)),
        pl.BlockSpec((1, _DH), lambda i: (0, 0)),
        pl.BlockSpec((1, _DH), lambda i: (0, 0)),
        pl.BlockSpec((_RB, 1), lambda i: (i, 0)),
        pl.BlockSpec((_DH, _DH), lambda i: (0, 0)),
        pl.BlockSpec((1, _DH), lambda i: (0, 0)),
        pl.BlockSpec((_DE, _DH), lambda i: (0, 0)),
        pl.BlockSpec((1, _DE), lambda i: (0, 0)),
    ],
    out_specs=pl.BlockSpec((_BG, _DE), lambda i: (0, 0)),
    out_shape=jax.ShapeDtypeStruct((_BG, _DE), jnp.float32),
    scratch_shapes=[
        pltpu.VMEM((_BG, _DH), jnp.float32),
        pltpu.VMEM((1, _BG), jnp.float32),
    ],
)


def kernel(x, edge_index, batch, Win, bin_, W1, b1, g1, be1, rm1, rv1,
           W2, b2, g2, be2, rm2, rv2, W3, b3, g3, be3, rm3, rv3,
           Wo1, bo1, Wo2, bo2):
    src = edge_index[0]
    dst = edge_index[1]
    padlen = _EP - _E
    srcp = jnp.concatenate([src, jnp.zeros((padlen,), jnp.int32)])
    dstp = jnp.concatenate([dst, jnp.full((padlen,), _JR, jnp.int32)])
    src2 = jnp.stack([srcp, srcp + _N]).reshape(_NC, _NS, _NBL, _K)
    dst2 = dstp.reshape(_NS, _NBL, _K)
    srcz = jnp.zeros((_EP,), jnp.int32)
    src0 = jnp.stack([srcz, srcz + _N]).reshape(_NC, _NS, _NBL, _K)
    ones_tab = jnp.ones((_NC * _N, _HALF), jnp.float32)

    r = lambda v: v.reshape(1, -1)

    degdump = _seg_call(ones_tab, src0, dst2)
    hw1, dis = _tc0_call(x, Win, r(bin_), W1, degdump)
    seg1 = _seg_call(hw1.reshape(_NC * _N, _HALF), src2, dst2)
    hw2 = _tcmid_call(seg1, hw1, dis, r(b1), r(g1), r(be1), r(rm1), r(rv1), W2)
    seg2 = _seg_call(hw2.reshape(_NC * _N, _HALF), src2, dst2)
    hw3 = _tcmid_call(seg2, hw2, dis, r(b2), r(g2), r(be2), r(rm2), r(rv2), W3)
    seg3 = _seg_call(hw3.reshape(_NC * _N, _HALF), src2, dst2)
    out = _tc3_call(seg3, hw3, dis, r(b3), r(g3), r(be3), r(rm3), r(rv3),
                    batch.reshape(_N, 1), Wo1, r(bo1), Wo2, r(bo2))
    return out


# speedup vs baseline: 6.5434x; 4.0350x over previous
import jax
import jax.numpy as jnp
from jax import lax
from jax.experimental import pallas as pl
from jax.experimental.pallas import tpu as pltpu
from jax.experimental.pallas import tpu_sc as plsc

_N = 50000
_E = 800000
_DIN = 128
_DH = 64
_DE = 32
_BG = 64

_NC, _NS = 2, 16
_K = 128
_NBL = 392
_NBD = 196
_EP = _NS * _NBL * _K
_JR = _N
_RA = 30336
_RS = 30328
_NQ = 2
_NOUT = _NQ * _RS
_RPT = _RA // _NS
_HALF = _DH // 2

_mesh = plsc.VectorSubcoreMesh(core_axis_name="c", subcore_axis_name="s",
                               num_cores=_NC, num_subcores=_NS)


def _fill2(ref, nrows, value):
    v = jnp.full((16,), value, jnp.float32)

    def body(r, carry):
        ref[r, pl.ds(0, 16)] = v
        ref[r, pl.ds(16, 16)] = v
        return carry

    lax.fori_loop(0, nrows, body, 0)


def _remap(idx_ref, base):
    def body(k, carry):
        v = idx_ref[pl.ds(k * 16, 16)]
        lo = v - base
        ok = (lo >= 0) & (lo < _RS)
        idx_ref[pl.ds(k * 16, 16)] = jnp.where(ok, lo, _RS)
        return carry

    lax.fori_loop(0, _K // 16, body, 0)


def _dump(acc, out, c, s, qbase):
    @pl.when(s < _NS - 1)
    def _():
        pltpu.sync_copy(acc.at[pl.ds(s * _RPT, _RPT)],
                        out.at[c, pl.ds(qbase + s * _RPT, _RPT)])

    @pl.when(s == _NS - 1)
    def _():
        pltpu.sync_copy(acc.at[pl.ds((_NS - 1) * _RPT, _RPT - 8)],
                        out.at[c, pl.ds(qbase + (_NS - 1) * _RPT, _RPT - 8)])


def _seg_body(table, src2, dst2, out, isa_, isb_, ida_, idb_, rowa, rowb, zb,
              acc, s_is_a, s_is_b, s_id_a, s_id_b, s_g_a, s_g_b):
    c = lax.axis_index("c")
    s = lax.axis_index("s")
    _fill2(zb, _RPT, 0.0)

    def istart(j, islot, dslot, si, sd):
        pltpu.make_async_copy(src2.at[c, s, j], islot, si).start()
        pltpu.make_async_copy(dst2.at[s, j], dslot, sd).start()

    def iwait_src(j, islot, si):
        pltpu.make_async_copy(src2.at[c, s, j], islot, si).wait()

    def iwait_dst(j, dslot, sd):
        pltpu.make_async_copy(dst2.at[s, j], dslot, sd).wait()

    def gstart(islot, rslot, sg):
        pltpu.make_async_copy(table.at[islot], rslot, sg).start()

    def gwait(islot, rslot, sg):
        pltpu.make_async_copy(table.at[islot], rslot, sg).wait()

    for q in range(_NQ):
        base = q * _RS
        pltpu.sync_copy(zb, acc.at[pl.ds(s * _RPT, _RPT)])
        plsc.subcore_barrier()

        istart(0, isa_, ida_, s_is_a, s_id_a)
        iwait_src(0, isa_, s_is_a)
        gstart(isa_, rowa, s_g_a)

        def body(jo, carry):
            j0 = 2 * jo
            istart(j0 + 1, isb_, idb_, s_is_b, s_id_b)
            gwait(isa_, rowa, s_g_a)
            iwait_src(j0 + 1, isb_, s_is_b)
            gstart(isb_, rowb, s_g_b)
            iwait_dst(j0, ida_, s_id_a)
            _remap(ida_, base)
            pltpu.sync_copy(rowa, acc.at[ida_], add=True)
            gwait(isb_, rowb, s_g_b)

            @pl.when(jo < _NBL // 2 - 1)
            def _():
                istart(j0 + 2, isa_, ida_, s_is_a, s_id_a)
                iwait_src(j0 + 2, isa_, s_is_a)
                gstart(isa_, rowa, s_g_a)

            iwait_dst(j0 + 1, idb_, s_id_b)
            _remap(idb_, base)
            pltpu.sync_copy(rowb, acc.at[idb_], add=True)
            return carry

        lax.fori_loop(0, _NBL // 2, body, 0)
        plsc.subcore_barrier()
        _dump(acc, out, c, s, base)


_seg_call = pl.kernel(
    _seg_body,
    out_type=jax.ShapeDtypeStruct((_NC, _NOUT, _HALF), jnp.float32),
    mesh=_mesh,
    compiler_params=pltpu.CompilerParams(use_tc_tiling_on_sc=False),
    scratch_types=[
        pltpu.VMEM((_K,), jnp.int32),
        pltpu.VMEM((_K,), jnp.int32),
        pltpu.VMEM((_K,), jnp.int32),
        pltpu.VMEM((_K,), jnp.int32),
        pltpu.VMEM((_K, _HALF), jnp.float32),
        pltpu.VMEM((_K, _HALF), jnp.float32),
        pltpu.VMEM((_RPT, _HALF), jnp.float32),
        pltpu.VMEM_SHARED((_RA, _HALF), jnp.float32),
        pltpu.SemaphoreType.DMA,
        pltpu.SemaphoreType.DMA,
        pltpu.SemaphoreType.DMA,
        pltpu.SemaphoreType.DMA,
        pltpu.SemaphoreType.DMA,
        pltpu.SemaphoreType.DMA,
    ],
)


_RB = 1000
_GRID = _N // _RB


def _mm_t(a, w):
    return lax.dot_general(a, w, (((1,), (1,)), ((), ())),
                           preferred_element_type=jnp.float32)


def _tc0_body(x_ref, win_ref, bin_ref, w1_ref, dd_ref, hw_ref, dis_ref):
    deg = dd_ref[0, :, 0:1] + 1.0
    dis = lax.rsqrt(deg)
    h0 = jnp.maximum(_mm_t(x_ref[...], win_ref[...]) + bin_ref[...], 0.0)
    hw = _mm_t(h0, w1_ref[...]) * dis
    hw_ref[0] = hw[:, :_HALF]
    hw_ref[1] = hw[:, _HALF:]
    dis_ref[...] = dis


_tc0_call = pl.pallas_call(
    _tc0_body,
    grid=(_GRID,),
    in_specs=[
        pl.BlockSpec((_RB, _DIN), lambda i: (i, 0)),
        pl.BlockSpec((_DH, _DIN), lambda i: (0, 0)),
        pl.BlockSpec((1, _DH), lambda i: (0, 0)),
        pl.BlockSpec((_DH, _DH), lambda i: (0, 0)),
        pl.BlockSpec((_NC, _RB, _HALF), lambda i: (0, i, 0)),
    ],
    out_specs=[
        pl.BlockSpec((_NC, _RB, _HALF), lambda i: (0, i, 0)),
        pl.BlockSpec((_RB, 1), lambda i: (i, 0)),
    ],
    out_shape=[
        jax.ShapeDtypeStruct((_NC, _N, _HALF), jnp.float32),
        jax.ShapeDtypeStruct((_N, 1), jnp.float32),
    ],
)


def _bn_relu(sd_ref, hwp_ref, dis_ref, b_ref, g_ref, be_ref, rm_ref, rv_ref):
    sc = g_ref[...] * lax.rsqrt(rv_ref[...] + 1e-5)
    tb = (b_ref[...] - rm_ref[...]) * sc + be_ref[...]
    seg = jnp.concatenate([sd_ref[0] + hwp_ref[0], sd_ref[1] + hwp_ref[1]],
                          axis=1) * dis_ref[...]
    return jnp.maximum(seg * sc + tb, 0.0)


def _tcmid_body(sd_ref, hwp_ref, dis_ref, b_ref, g_ref, be_ref, rm_ref,
                rv_ref, wn_ref, hw_ref):
    h = _bn_relu(sd_ref, hwp_ref, dis_ref, b_ref, g_ref, be_ref, rm_ref, rv_ref)
    hw = _mm_t(h, wn_ref[...]) * dis_ref[...]
    hw_ref[0] = hw[:, :_HALF]
    hw_ref[1] = hw[:, _HALF:]


_tcmid_call = pl.pallas_call(
    _tcmid_body,
    grid=(_GRID,),
    in_specs=[
        pl.BlockSpec((_NC, _RB, _HALF), lambda i: (0, i, 0)),
        pl.BlockSpec((_NC, _RB, _HALF), lambda i: (0, i, 0)),
        pl.BlockSpec((_RB, 1), lambda i: (i, 0)),
        pl.BlockSpec((1, _DH), lambda i: (0, 0)),
        pl.BlockSpec((1, _DH), lambda i: (0, 0)),
        pl.BlockSpec((1, _DH), lambda i: (0, 0)),
        pl.BlockSpec((1, _DH), lambda i: (0, 0)),
        pl.BlockSpec((1, _DH), lambda i: (0, 0)),
        pl.BlockSpec((_DH, _DH), lambda i: (0, 0)),
    ],
    out_specs=pl.BlockSpec((_NC, _RB, _HALF), lambda i: (0, i, 0)),
    out_shape=jax.ShapeDtypeStruct((_NC, _N, _HALF), jnp.float32),
)


def _tc3_body(sd_ref, hwp_ref, dis_ref, b_ref, g_ref, be_ref, rm_ref, rv_ref,
              batch_ref, wo1_ref, bo1_ref, wo2_ref, bo2_ref, out_ref,
              accp, accc):
    i = pl.program_id(0)

    @pl.when(i == 0)
    def _():
        accp[...] = jnp.zeros_like(accp)
        accc[...] = jnp.zeros_like(accc)

    h = _bn_relu(sd_ref, hwp_ref, dis_ref, b_ref, g_ref, be_ref, rm_ref, rv_ref)
    gid = lax.broadcasted_iota(jnp.int32, (_RB, _BG), 1)
    oh = (batch_ref[...] == gid).astype(jnp.float32)
    accp[...] += lax.dot_general(oh, h, (((0,), (0,)), ((), ())),
                                 preferred_element_type=jnp.float32)
    accc[...] += jnp.sum(oh, axis=0, keepdims=True)

    @pl.when(i == _GRID - 1)
    def _():
        cnt = jnp.reshape(jnp.maximum(accc[...], 1.0), (_BG, 1))
        pooled = accp[...] / cnt
        hid = jnp.maximum(_mm_t(pooled, wo1_ref[...]) + bo1_ref[...], 0.0)
        out_ref[...] = _mm_t(hid, wo2_ref[...]) + bo2_ref[...]


_tc3_call = pl.pallas_call(
    _tc3_body,
    grid=(_GRID,),
    in_specs=[
        pl.BlockSpec((_NC, _RB, _HALF), lambda i: (0, i, 0)),
        pl.BlockSpec((_NC, _RB, _HALF), lambda i: (0, i, 0)),
        pl.BlockSpec((_RB, 1), lambda i: (i, 0)),
        pl.BlockSpec((1, _DH), lambda i: (0, 0)),
        pl.BlockSpec((1, _DH), lambda i: (0, 0)),
        pl.BlockSpec((1, _DH), lambda i: (0, 0)),
        pl.BlockSpec((1, _DH), lambda i: (0, 0)),
        pl.BlockSpec((1, _DH), lambda i: (0, 0)),
        pl.BlockSpec((_RB, 1), lambda i: (i, 0)),
        pl.BlockSpec((_DH, _DH), lambda i: (0, 0)),
        pl.BlockSpec((1, _DH), lambda i: (0, 0)),
        pl.BlockSpec((_DE, _DH), lambda i: (0, 0)),
        pl.BlockSpec((1, _DE), lambda i: (0, 0)),
    ],
    out_specs=pl.BlockSpec((_BG, _DE), lambda i: (0, 0)),
    out_shape=jax.ShapeDtypeStruct((_BG, _DE), jnp.float32),
    scratch_shapes=[
        pltpu.VMEM((_BG, _DH), jnp.float32),
        pltpu.VMEM((1, _BG), jnp.float32),
    ],
)


def kernel(x, edge_index, batch, Win, bin_, W1, b1, g1, be1, rm1, rv1,
           W2, b2, g2, be2, rm2, rv2, W3, b3, g3, be3, rm3, rv3,
           Wo1, bo1, Wo2, bo2):
    src = edge_index[0]
    dst = edge_index[1]
    padlen = _EP - _E
    srcp = jnp.concatenate([src, jnp.zeros((padlen,), jnp.int32)])
    dstp = jnp.concatenate([dst, jnp.full((padlen,), _JR, jnp.int32)])
    src2 = jnp.stack([srcp, srcp + _N]).reshape(_NC, _NS, _NBL, _K)
    dst2 = dstp.reshape(_NS, _NBL, _K)
    ones_tab = jnp.ones((_NC * _N, _HALF), jnp.float32)

    r = lambda v: v.reshape(1, -1)

    degdump = _seg_call(ones_tab, src2, dst2)
    hw1, dis = _tc0_call(x, Win, r(bin_), W1, degdump)
    seg1 = _seg_call(hw1.reshape(_NC * _N, _HALF), src2, dst2)
    hw2 = _tcmid_call(seg1, hw1, dis, r(b1), r(g1), r(be1), r(rm1), r(rv1), W2)
    seg2 = _seg_call(hw2.reshape(_NC * _N, _HALF), src2, dst2)
    hw3 = _tcmid_call(seg2, hw2, dis, r(b2), r(g2), r(be2), r(rm2), r(rv2), W3)
    seg3 = _seg_call(hw3.reshape(_NC * _N, _HALF), src2, dst2)
    out = _tc3_call(seg3, hw3, dis, r(b3), r(g3), r(be3), r(rm3), r(rv3),
                    batch.reshape(_N, 1), Wo1, r(bo1), Wo2, r(bo2))
    return out


# speedup vs baseline: 6.7116x; 1.0257x over previous
import jax
import jax.numpy as jnp
from jax import lax
from jax.experimental import pallas as pl
from jax.experimental.pallas import tpu as pltpu
from jax.experimental.pallas import tpu_sc as plsc

_N = 50000
_E = 800000
_DIN = 128
_DH = 64
_DE = 32
_BG = 64

_NC, _NS = 2, 16
_K = 128
_NBL = 392
_NBD = 196
_EP = _NS * _NBL * _K
_JR = _N
_RA = 28288
_RS = 28280
_NQ = 2
_NOUT = _NQ * _RS
_RPT = _RA // _NS
_HALF = _DH // 2

_mesh = plsc.VectorSubcoreMesh(core_axis_name="c", subcore_axis_name="s",
                               num_cores=_NC, num_subcores=_NS)


def _fill2(ref, nrows, value):
    v = jnp.full((16,), value, jnp.float32)

    def body(r, carry):
        ref[r, pl.ds(0, 16)] = v
        ref[r, pl.ds(16, 16)] = v
        return carry

    lax.fori_loop(0, nrows, body, 0)


def _remap(idx_ref, base):
    for k in range(_K // 16):
        v = idx_ref[pl.ds(k * 16, 16)]
        lo = v - base
        ok = (lo >= 0) & (lo < _RS)
        idx_ref[pl.ds(k * 16, 16)] = jnp.where(ok, lo, _RS)


def _dump(acc, out, c, s, qbase):
    @pl.when(s < _NS - 1)
    def _():
        pltpu.sync_copy(acc.at[pl.ds(s * _RPT, _RPT)],
                        out.at[c, pl.ds(qbase + s * _RPT, _RPT)])

    @pl.when(s == _NS - 1)
    def _():
        pltpu.sync_copy(acc.at[pl.ds((_NS - 1) * _RPT, _RPT - 8)],
                        out.at[c, pl.ds(qbase + (_NS - 1) * _RPT, _RPT - 8)])


def _seg_body(table, src2, dst2, out,
              is0, is1, is2, is3, id0, id1, id2, id3, r0, r1, r2, r3, zb, acc,
              sis0, sis1, sis2, sis3, sid0, sid1, sid2, sid3,
              sg0, sg1, sg2, sg3, ssc0, ssc1, ssc2, ssc3):
    c = lax.axis_index("c")
    s = lax.axis_index("s")
    iss = (is0, is1, is2, is3)
    ids = (id0, id1, id2, id3)
    rows = (r0, r1, r2, r3)
    sis = (sis0, sis1, sis2, sis3)
    sid = (sid0, sid1, sid2, sid3)
    sg = (sg0, sg1, sg2, sg3)
    ssc = (ssc0, ssc1, ssc2, ssc3)
    _fill2(zb, _RPT, 0.0)
    nb4 = _NBL // 4

    def istart_src(j, b):
        pltpu.make_async_copy(src2.at[c, s, j], iss[b], sis[b]).start()

    def iwait_src(j, b):
        pltpu.make_async_copy(src2.at[c, s, j], iss[b], sis[b]).wait()

    def istart_dst(j, b):
        pltpu.make_async_copy(dst2.at[s, j], ids[b], sid[b]).start()

    def iwait_dst(j, b):
        pltpu.make_async_copy(dst2.at[s, j], ids[b], sid[b]).wait()

    def gstart(b):
        pltpu.make_async_copy(table.at[iss[b]], rows[b], sg[b]).start()

    def gwait(b):
        pltpu.make_async_copy(table.at[iss[b]], rows[b], sg[b]).wait()

    def scstart(b):
        pltpu.make_async_copy(rows[b], acc.at[ids[b]], ssc[b]).start(add=True)

    def scwait(b):
        pltpu.make_async_copy(rows[b], acc.at[ids[b]], ssc[b]).wait()

    for q in range(_NQ):
        base = q * _RS
        pltpu.sync_copy(zb, acc.at[pl.ds(s * _RPT, _RPT)])
        plsc.subcore_barrier()

        for b in range(4):
            istart_src(b, b)
        for b in range(2):
            istart_dst(b, b)
        for b in range(2):
            iwait_src(b, b)
            gstart(b)
        iwait_dst(0, 0)
        _remap(id0, base)

        def body(jo, carry):
            for b in range(4):
                j = 4 * jo + b
                b1 = (b + 1) % 4
                b2 = (b + 2) % 4
                gwait(b)
                scstart(b)

                @pl.when(jo < nb4 - 1)
                def _():
                    istart_src(j + 4, b)

                if b < 2:
                    @pl.when(jo > 0)
                    def _():
                        scwait(b2)
                    istart_dst(j + 2, b2)
                    iwait_src(j + 2, b2)
                    gstart(b2)
                else:
                    scwait(b2)

                    @pl.when(jo < nb4 - 1)
                    def _():
                        istart_dst(j + 2, b2)
                        iwait_src(j + 2, b2)
                        gstart(b2)

                if b < 3:
                    iwait_dst(j + 1, b1)
                    _remap(ids[b1], base)
                else:
                    @pl.when(jo < nb4 - 1)
                    def _():
                        iwait_dst(j + 1, b1)
                        _remap(ids[b1], base)
            return carry

        lax.fori_loop(0, nb4, body, 0)
        scwait(2)
        scwait(3)
        plsc.subcore_barrier()
        _dump(acc, out, c, s, base)


_seg_call = pl.kernel(
    _seg_body,
    out_type=jax.ShapeDtypeStruct((_NC, _NOUT, _HALF), jnp.float32),
    mesh=_mesh,
    compiler_params=pltpu.CompilerParams(use_tc_tiling_on_sc=False),
    scratch_types=(
        [pltpu.VMEM((_K,), jnp.int32) for _ in range(8)]
        + [pltpu.VMEM((_K, _HALF), jnp.float32) for _ in range(4)]
        + [pltpu.VMEM((_RPT, _HALF), jnp.float32),
           pltpu.VMEM_SHARED((_RA, _HALF), jnp.float32)]
        + [pltpu.SemaphoreType.DMA for _ in range(16)]
    ),
)


_RB = 1000
_GRID = _N // _RB


def _mm_t(a, w):
    return lax.dot_general(a, w, (((1,), (1,)), ((), ())),
                           preferred_element_type=jnp.float32)


def _tc0_body(x_ref, win_ref, bin_ref, w1_ref, dd_ref, hw_ref, dis_ref):
    deg = dd_ref[0, :, 0:1] + 1.0
    dis = lax.rsqrt(deg)
    h0 = jnp.maximum(_mm_t(x_ref[...], win_ref[...]) + bin_ref[...], 0.0)
    hw = _mm_t(h0, w1_ref[...]) * dis
    hw_ref[0] = hw[:, :_HALF]
    hw_ref[1] = hw[:, _HALF:]
    dis_ref[...] = dis


_tc0_call = pl.pallas_call(
    _tc0_body,
    grid=(_GRID,),
    in_specs=[
        pl.BlockSpec((_RB, _DIN), lambda i: (i, 0)),
        pl.BlockSpec((_DH, _DIN), lambda i: (0, 0)),
        pl.BlockSpec((1, _DH), lambda i: (0, 0)),
        pl.BlockSpec((_DH, _DH), lambda i: (0, 0)),
        pl.BlockSpec((_NC, _RB, _HALF), lambda i: (0, i, 0)),
    ],
    out_specs=[
        pl.BlockSpec((_NC, _RB, _HALF), lambda i: (0, i, 0)),
        pl.BlockSpec((_RB, 1), lambda i: (i, 0)),
    ],
    out_shape=[
        jax.ShapeDtypeStruct((_NC, _N, _HALF), jnp.float32),
        jax.ShapeDtypeStruct((_N, 1), jnp.float32),
    ],
)


def _bn_relu(sd_ref, hwp_ref, dis_ref, b_ref, g_ref, be_ref, rm_ref, rv_ref):
    sc = g_ref[...] * lax.rsqrt(rv_ref[...] + 1e-5)
    tb = (b_ref[...] - rm_ref[...]) * sc + be_ref[...]
    seg = jnp.concatenate([sd_ref[0] + hwp_ref[0], sd_ref[1] + hwp_ref[1]],
                          axis=1) * dis_ref[...]
    return jnp.maximum(seg * sc + tb, 0.0)


def _tcmid_body(sd_ref, hwp_ref, dis_ref, b_ref, g_ref, be_ref, rm_ref,
                rv_ref, wn_ref, hw_ref):
    h = _bn_relu(sd_ref, hwp_ref, dis_ref, b_ref, g_ref, be_ref, rm_ref, rv_ref)
    hw = _mm_t(h, wn_ref[...]) * dis_ref[...]
    hw_ref[0] = hw[:, :_HALF]
    hw_ref[1] = hw[:, _HALF:]


_tcmid_call = pl.pallas_call(
    _tcmid_body,
    grid=(_GRID,),
    in_specs=[
        pl.BlockSpec((_NC, _RB, _HALF), lambda i: (0, i, 0)),
        pl.BlockSpec((_NC, _RB, _HALF), lambda i: (0, i, 0)),
        pl.BlockSpec((_RB, 1), lambda i: (i, 0)),
        pl.BlockSpec((1, _DH), lambda i: (0, 0)),
        pl.BlockSpec((1, _DH), lambda i: (0, 0)),
        pl.BlockSpec((1, _DH), lambda i: (0, 0)),
        pl.BlockSpec((1, _DH), lambda i: (0, 0)),
        pl.BlockSpec((1, _DH), lambda i: (0, 0)),
        pl.BlockSpec((_DH, _DH), lambda i: (0, 0)),
    ],
    out_specs=pl.BlockSpec((_NC, _RB, _HALF), lambda i: (0, i, 0)),
    out_shape=jax.ShapeDtypeStruct((_NC, _N, _HALF), jnp.float32),
)


def _tc3_body(sd_ref, hwp_ref, dis_ref, b_ref, g_ref, be_ref, rm_ref, rv_ref,
              batch_ref, wo1_ref, bo1_ref, wo2_ref, bo2_ref, out_ref,
              accp, accc):
    i = pl.program_id(0)

    @pl.when(i == 0)
    def _():
        accp[...] = jnp.zeros_like(accp)
        accc[...] = jnp.zeros_like(accc)

    h = _bn_relu(sd_ref, hwp_ref, dis_ref, b_ref, g_ref, be_ref, rm_ref, rv_ref)
    gid = lax.broadcasted_iota(jnp.int32, (_RB, _BG), 1)
    oh = (batch_ref[...] == gid).astype(jnp.float32)
    accp[...] += lax.dot_general(oh, h, (((0,), (0,)), ((), ())),
                                 preferred_element_type=jnp.float32)
    accc[...] += jnp.sum(oh, axis=0, keepdims=True)

    @pl.when(i == _GRID - 1)
    def _():
        cnt = jnp.reshape(jnp.maximum(accc[...], 1.0), (_BG, 1))
        pooled = accp[...] / cnt
        hid = jnp.maximum(_mm_t(pooled, wo1_ref[...]) + bo1_ref[...], 0.0)
        out_ref[...] = _mm_t(hid, wo2_ref[...]) + bo2_ref[...]


_tc3_call = pl.pallas_call(
    _tc3_body,
    grid=(_GRID,),
    in_specs=[
        pl.BlockSpec((_NC, _RB, _HALF), lambda i: (0, i, 0)),
        pl.BlockSpec((_NC, _RB, _HALF), lambda i: (0, i, 0)),
        pl.BlockSpec((_RB, 1), lambda i: (i, 0)),
        pl.BlockSpec((1, _DH), lambda i: (0, 0)),
        pl.BlockSpec((1, _DH), lambda i: (0, 0)),
        pl.BlockSpec((1, _DH), lambda i: (0, 0)),
        pl.BlockSpec((1, _DH), lambda i: (0, 0)),
        pl.BlockSpec((1, _DH), lambda i: (0, 0)),
        pl.BlockSpec((_RB, 1), lambda i: (i, 0)),
        pl.BlockSpec((_DH, _DH), lambda i: (0, 0)),
        pl.BlockSpec((1, _DH), lambda i: (0, 0)),
        pl.BlockSpec((_DE, _DH), lambda i: (0, 0)),
        pl.BlockSpec((1, _DE), lambda i: (0, 0)),
    ],
    out_specs=pl.BlockSpec((_BG, _DE), lambda i: (0, 0)),
    out_shape=jax.ShapeDtypeStruct((_BG, _DE), jnp.float32),
    scratch_shapes=[
        pltpu.VMEM((_BG, _DH), jnp.float32),
        pltpu.VMEM((1, _BG), jnp.float32),
    ],
)


def kernel(x, edge_index, batch, Win, bin_, W1, b1, g1, be1, rm1, rv1,
           W2, b2, g2, be2, rm2, rv2, W3, b3, g3, be3, rm3, rv3,
           Wo1, bo1, Wo2, bo2):
    src = edge_index[0]
    dst = edge_index[1]
    padlen = _EP - _E
    srcp = jnp.concatenate([src, jnp.zeros((padlen,), jnp.int32)])
    dstp = jnp.concatenate([dst, jnp.full((padlen,), _JR, jnp.int32)])
    src2 = jnp.stack([srcp, srcp + _N]).reshape(_NC, _NS, _NBL, _K)
    dst2 = dstp.reshape(_NS, _NBL, _K)
    ones_tab = jnp.ones((_NC * _N, _HALF), jnp.float32)

    r = lambda v: v.reshape(1, -1)

    degdump = _seg_call(ones_tab, src2, dst2)
    hw1, dis = _tc0_call(x, Win, r(bin_), W1, degdump)
    seg1 = _seg_call(hw1.reshape(_NC * _N, _HALF), src2, dst2)
    hw2 = _tcmid_call(seg1, hw1, dis, r(b1), r(g1), r(be1), r(rm1), r(rv1), W2)
    seg2 = _seg_call(hw2.reshape(_NC * _N, _HALF), src2, dst2)
    hw3 = _tcmid_call(seg2, hw2, dis, r(b2), r(g2), r(be2), r(rm2), r(rv2), W3)
    seg3 = _seg_call(hw3.reshape(_NC * _N, _HALF), src2, dst2)
    out = _tc3_call(seg3, hw3, dis, r(b3), r(g3), r(be3), r(rm3), r(rv3),
                    batch.reshape(_N, 1), Wo1, r(bo1), Wo2, r(bo2))
    return out


# speedup vs baseline: 13.2454x; 1.9735x over previous
import jax
import jax.numpy as jnp
from jax import lax
from jax.experimental import pallas as pl
from jax.experimental.pallas import tpu as pltpu
from jax.experimental.pallas import tpu_sc as plsc

_N = 50000
_E = 800000
_DIN = 128
_DH = 64
_DE = 32
_BG = 64

_NC, _NS = 2, 16
_K = 128
_NBL = 392
_EP = _NS * _NBL * _K
_JR = _N
_FW = 16
_NG = 4
_NP = 2
_RA = 50048
_RPT = _RA // _NS

_mesh = plsc.VectorSubcoreMesh(core_axis_name="c", subcore_axis_name="s",
                               num_cores=_NC, num_subcores=_NS)


def _fill(ref, nrows, value):
    v = jnp.full((16,), value, jnp.float32)

    def body(r, carry):
        ref[r, pl.ds(0, 16)] = v
        return carry

    lax.fori_loop(0, nrows, body, 0)


def _seg_body(table, src2, dst2, out,
              is0, is1, is2, is3, id0, id1, id2, id3, r0, r1, r2, r3, zb, acc,
              sis0, sis1, sis2, sis3, sid0, sid1, sid2, sid3,
              sg0, sg1, sg2, sg3, ssc0, ssc1, ssc2, ssc3):
    c = lax.axis_index("c")
    s = lax.axis_index("s")
    iss = (is0, is1, is2, is3)
    ids = (id0, id1, id2, id3)
    rows = (r0, r1, r2, r3)
    sis = (sis0, sis1, sis2, sis3)
    sid = (sid0, sid1, sid2, sid3)
    sg = (sg0, sg1, sg2, sg3)
    ssc = (ssc0, ssc1, ssc2, ssc3)
    _fill(zb, _RPT, 0.0)
    nb4 = _NBL // 4

    def make(p):
        def istart_src(j, b):
            pltpu.make_async_copy(src2.at[p, c, s, j], iss[b], sis[b]).start()

        def iwait_src(j, b):
            pltpu.make_async_copy(src2.at[p, c, s, j], iss[b], sis[b]).wait()

        def istart_dst(j, b):
            pltpu.make_async_copy(dst2.at[s, j], ids[b], sid[b]).start()

        def iwait_dst(j, b):
            pltpu.make_async_copy(dst2.at[s, j], ids[b], sid[b]).wait()

        def gstart(b):
            pltpu.make_async_copy(table.at[iss[b]], rows[b], sg[b]).start()

        def gwait(b):
            pltpu.make_async_copy(table.at[iss[b]], rows[b], sg[b]).wait()

        def scstart(b):
            pltpu.make_async_copy(rows[b], acc.at[ids[b]], ssc[b]).start(add=True)

        def scwait(b):
            pltpu.make_async_copy(rows[b], acc.at[ids[b]], ssc[b]).wait()

        return (istart_src, iwait_src, istart_dst, iwait_dst, gstart, gwait,
                scstart, scwait)

    for p in range(_NP):
        (istart_src, iwait_src, istart_dst, iwait_dst, gstart, gwait,
         scstart, scwait) = make(p)
        pltpu.sync_copy(zb, acc.at[pl.ds(s * _RPT, _RPT)])
        plsc.subcore_barrier()

        for b in range(4):
            istart_src(b, b)
        for b in range(2):
            istart_dst(b, b)
        for b in range(2):
            iwait_src(b, b)
            gstart(b)

        def body(jo, carry):
            for b in range(4):
                j = 4 * jo + b
                b2 = (b + 2) % 4
                gwait(b)
                iwait_dst(j, b)
                scstart(b)

                @pl.when(jo < nb4 - 1)
                def _():
                    istart_src(j + 4, b)

                if b < 2:
                    @pl.when(jo > 0)
                    def _():
                        scwait(b2)
                    istart_dst(j + 2, b2)
                    iwait_src(j + 2, b2)
                    gstart(b2)
                else:
                    scwait(b2)

                    @pl.when(jo < nb4 - 1)
                    def _():
                        istart_dst(j + 2, b2)
                        iwait_src(j + 2, b2)
                        gstart(b2)
            return carry

        lax.fori_loop(0, nb4, body, 0)
        scwait(2)
        scwait(3)
        plsc.subcore_barrier()
        pltpu.sync_copy(acc.at[pl.ds(s * _RPT, _RPT)],
                        out.at[2 * p + c, pl.ds(s * _RPT, _RPT)])


_seg_call = pl.kernel(
    _seg_body,
    out_type=jax.ShapeDtypeStruct((_NG, _RA, _FW), jnp.float32),
    mesh=_mesh,
    compiler_params=pltpu.CompilerParams(use_tc_tiling_on_sc=False),
    scratch_types=(
        [pltpu.VMEM((_K,), jnp.int32) for _ in range(8)]
        + [pltpu.VMEM((_K, _FW), jnp.float32) for _ in range(4)]
        + [pltpu.VMEM((_RPT, _FW), jnp.float32),
           pltpu.VMEM_SHARED((_RA, _FW), jnp.float32)]
        + [pltpu.SemaphoreType.DMA for _ in range(16)]
    ),
)


_RB = 1000
_GRID = _N // _RB


def _mm_t(a, w):
    return lax.dot_general(a, w, (((1,), (1,)), ((), ())),
                           preferred_element_type=jnp.float32)


def _split_groups(hw_ref, hw):
    for g in range(_NG):
        hw_ref[g] = hw[:, g * _FW:(g + 1) * _FW]


def _tc0_body(x_ref, win_ref, bin_ref, w1_ref, dd_ref, hw_ref, dis_ref):
    deg = dd_ref[0, :, 0:1] + 1.0
    dis = lax.rsqrt(deg)
    h0 = jnp.maximum(_mm_t(x_ref[...], win_ref[...]) + bin_ref[...], 0.0)
    hw = _mm_t(h0, w1_ref[...]) * dis
    _split_groups(hw_ref, hw)
    dis_ref[...] = dis


_tc0_call = pl.pallas_call(
    _tc0_body,
    grid=(_GRID,),
    in_specs=[
        pl.BlockSpec((_RB, _DIN), lambda i: (i, 0)),
        pl.BlockSpec((_DH, _DIN), lambda i: (0, 0)),
        pl.BlockSpec((1, _DH), lambda i: (0, 0)),
        pl.BlockSpec((_DH, _DH), lambda i: (0, 0)),
        pl.BlockSpec((_NG, _RB, _FW), lambda i: (0, i, 0)),
    ],
    out_specs=[
        pl.BlockSpec((_NG, _RB, _FW), lambda i: (0, i, 0)),
        pl.BlockSpec((_RB, 1), lambda i: (i, 0)),
    ],
    out_shape=[
        jax.ShapeDtypeStruct((_NG, _N, _FW), jnp.float32),
        jax.ShapeDtypeStruct((_N, 1), jnp.float32),
    ],
)


def _bn_relu(sd_ref, hwp_ref, dis_ref, b_ref, g_ref, be_ref, rm_ref, rv_ref):
    sc = g_ref[...] * lax.rsqrt(rv_ref[...] + 1e-5)
    tb = (b_ref[...] - rm_ref[...]) * sc + be_ref[...]
    seg = jnp.concatenate([sd_ref[g] + hwp_ref[g] for g in range(_NG)],
                          axis=1) * dis_ref[...]
    return jnp.maximum(seg * sc + tb, 0.0)


def _tcmid_body(sd_ref, hwp_ref, dis_ref, b_ref, g_ref, be_ref, rm_ref,
                rv_ref, wn_ref, hw_ref):
    h = _bn_relu(sd_ref, hwp_ref, dis_ref, b_ref, g_ref, be_ref, rm_ref, rv_ref)
    hw = _mm_t(h, wn_ref[...]) * dis_ref[...]
    _split_groups(hw_ref, hw)


_tcmid_call = pl.pallas_call(
    _tcmid_body,
    grid=(_GRID,),
    in_specs=[
        pl.BlockSpec((_NG, _RB, _FW), lambda i: (0, i, 0)),
        pl.BlockSpec((_NG, _RB, _FW), lambda i: (0, i, 0)),
        pl.BlockSpec((_RB, 1), lambda i: (i, 0)),
        pl.BlockSpec((1, _DH), lambda i: (0, 0)),
        pl.BlockSpec((1, _DH), lambda i: (0, 0)),
        pl.BlockSpec((1, _DH), lambda i: (0, 0)),
        pl.BlockSpec((1, _DH), lambda i: (0, 0)),
        pl.BlockSpec((1, _DH), lambda i: (0, 0)),
        pl.BlockSpec((_DH, _DH), lambda i: (0, 0)),
    ],
    out_specs=pl.BlockSpec((_NG, _RB, _FW), lambda i: (0, i, 0)),
    out_shape=jax.ShapeDtypeStruct((_NG, _N, _FW), jnp.float32),
)


def _tc3_body(sd_ref, hwp_ref, dis_ref, b_ref, g_ref, be_ref, rm_ref, rv_ref,
              batch_ref, wo1_ref, bo1_ref, wo2_ref, bo2_ref, out_ref,
              accp, accc):
    i = pl.program_id(0)

    @pl.when(i == 0)
    def _():
        accp[...] = jnp.zeros_like(accp)
        accc[...] = jnp.zeros_like(accc)

    h = _bn_relu(sd_ref, hwp_ref, dis_ref, b_ref, g_ref, be_ref, rm_ref, rv_ref)
    gid = lax.broadcasted_iota(jnp.int32, (_RB, _BG), 1)
    oh = (batch_ref[...] == gid).astype(jnp.float32)
    accp[...] += lax.dot_general(oh, h, (((0,), (0,)), ((), ())),
                                 preferred_element_type=jnp.float32)
    accc[...] += jnp.sum(oh, axis=0, keepdims=True)

    @pl.when(i == _GRID - 1)
    def _():
        cnt = jnp.reshape(jnp.maximum(accc[...], 1.0), (_BG, 1))
        pooled = accp[...] / cnt
        hid = jnp.maximum(_mm_t(pooled, wo1_ref[...]) + bo1_ref[...], 0.0)
        out_ref[...] = _mm_t(hid, wo2_ref[...]) + bo2_ref[...]


_tc3_call = pl.pallas_call(
    _tc3_body,
    grid=(_GRID,),
    in_specs=[
        pl.BlockSpec((_NG, _RB, _FW), lambda i: (0, i, 0)),
        pl.BlockSpec((_NG, _RB, _FW), lambda i: (0, i, 0)),
        pl.BlockSpec((_RB, 1), lambda i: (i, 0)),
        pl.BlockSpec((1, _DH), lambda i: (0, 0)),
        pl.BlockSpec((1, _DH), lambda i: (0, 0)),
        pl.BlockSpec((1, _DH), lambda i: (0, 0)),
        pl.BlockSpec((1, _DH), lambda i: (0, 0)),
        pl.BlockSpec((1, _DH), lambda i: (0, 0)),
        pl.BlockSpec((_RB, 1), lambda i: (i, 0)),
        pl.BlockSpec((_DH, _DH), lambda i: (0, 0)),
        pl.BlockSpec((1, _DH), lambda i: (0, 0)),
        pl.BlockSpec((_DE, _DH), lambda i: (0, 0)),
        pl.BlockSpec((1, _DE), lambda i: (0, 0)),
    ],
    out_specs=pl.BlockSpec((_BG, _DE), lambda i: (0, 0)),
    out_shape=jax.ShapeDtypeStruct((_BG, _DE), jnp.float32),
    scratch_shapes=[
        pltpu.VMEM((_BG, _DH), jnp.float32),
        pltpu.VMEM((1, _BG), jnp.float32),
    ],
)


def kernel(x, edge_index, batch, Win, bin_, W1, b1, g1, be1, rm1, rv1,
           W2, b2, g2, be2, rm2, rv2, W3, b3, g3, be3, rm3, rv3,
           Wo1, bo1, Wo2, bo2):
    src = edge_index[0]
    dst = edge_index[1]
    padlen = _EP - _E
    srcp = jnp.concatenate([src, jnp.zeros((padlen,), jnp.int32)])
    dstp = jnp.concatenate([dst, jnp.full((padlen,), _JR, jnp.int32)])
    src2 = jnp.stack([srcp + g * _N for g in range(_NG)]).reshape(
        _NP, _NC, _NS, _NBL, _K)
    dst2 = dstp.reshape(_NS, _NBL, _K)
    ones_tab = jnp.ones((_NG * _N, _FW), jnp.float32)

    r = lambda v: v.reshape(1, -1)

    degdump = _seg_call(ones_tab, src2, dst2)
    hw1, dis = _tc0_call(x, Win, r(bin_), W1, degdump)
    seg1 = _seg_call(hw1.reshape(_NG * _N, _FW), src2, dst2)
    hw2 = _tcmid_call(seg1, hw1, dis, r(b1), r(g1), r(be1), r(rm1), r(rv1), W2)
    seg2 = _seg_call(hw2.reshape(_NG * _N, _FW), src2, dst2)
    hw3 = _tcmid_call(seg2, hw2, dis, r(b2), r(g2), r(be2), r(rm2), r(rv2), W3)
    seg3 = _seg_call(hw3.reshape(_NG * _N, _FW), src2, dst2)
    out = _tc3_call(seg3, hw3, dis, r(b3), r(g3), r(be3), r(rm3), r(rv3),
                    batch.reshape(_N, 1), Wo1, r(bo1), Wo2, r(bo2))
    return out


# speedup vs baseline: 15.4778x; 1.1685x over previous
import jax
import jax.numpy as jnp
from jax import lax
from jax.experimental import pallas as pl
from jax.experimental.pallas import tpu as pltpu
from jax.experimental.pallas import tpu_sc as plsc

_N = 50000
_E = 800000
_DIN = 128
_DH = 64
_DE = 32
_BG = 64

_NC, _NS = 2, 16
_K = 128
_NBL = 392
_EP = _NS * _NBL * _K
_JR = _N
_FW = 16
_NG = 4
_NP = 2
_RA = 50048
_RPT = _RA // _NS

_mesh = plsc.VectorSubcoreMesh(core_axis_name="c", subcore_axis_name="s",
                               num_cores=_NC, num_subcores=_NS)


def _fill(ref, nrows, value):
    v = jnp.full((16,), value, jnp.float32)

    def body(r, carry):
        ref[r, pl.ds(0, 16)] = v
        return carry

    lax.fori_loop(0, nrows, body, 0)


def _seg_body(table, src2, dst2, out,
              is0, is1, is2, is3, id0, id1, id2, id3, r0, r1, r2, r3, zb, acc,
              sis0, sis1, sis2, sis3, sid0, sid1, sid2, sid3,
              sg0, sg1, sg2, sg3, ssc0, ssc1, ssc2, ssc3):
    c = lax.axis_index("c")
    s = lax.axis_index("s")
    iss = (is0, is1, is2, is3)
    ids = (id0, id1, id2, id3)
    rows = (r0, r1, r2, r3)
    sis = (sis0, sis1, sis2, sis3)
    sid = (sid0, sid1, sid2, sid3)
    sg = (sg0, sg1, sg2, sg3)
    ssc = (ssc0, ssc1, ssc2, ssc3)
    _fill(zb, _RPT, 0.0)
    nb4 = _NBL // 4

    def make(p):
        def istart_src(j, b):
            pltpu.make_async_copy(src2.at[p, c, s, j], iss[b], sis[b]).start()

        def iwait_src(j, b):
            pltpu.make_async_copy(src2.at[p, c, s, j], iss[b], sis[b]).wait()

        def istart_dst(j, b):
            pltpu.make_async_copy(dst2.at[s, j], ids[b], sid[b]).start()

        def iwait_dst(j, b):
            pltpu.make_async_copy(dst2.at[s, j], ids[b], sid[b]).wait()

        def gstart(b):
            pltpu.make_async_copy(table.at[iss[b]], rows[b], sg[b]).start()

        def gwait(b):
            pltpu.make_async_copy(table.at[iss[b]], rows[b], sg[b]).wait()

        def scstart(b):
            pltpu.make_async_copy(rows[b], acc.at[ids[b]], ssc[b]).start(add=True)

        def scwait(b):
            pltpu.make_async_copy(rows[b], acc.at[ids[b]], ssc[b]).wait()

        return (istart_src, iwait_src, istart_dst, iwait_dst, gstart, gwait,
                scstart, scwait)

    for p in range(_NP):
        (istart_src, iwait_src, istart_dst, iwait_dst, gstart, gwait,
         scstart, scwait) = make(p)
        pltpu.sync_copy(zb, acc.at[pl.ds(s * _RPT, _RPT)])
        plsc.subcore_barrier()

        for b in range(4):
            istart_src(b, b)
        for b in range(2):
            istart_dst(b, b)
        for b in range(2):
            iwait_src(b, b)
            gstart(b)

        def body(jo, carry):
            for b in range(4):
                j = 4 * jo + b
                b2 = (b + 2) % 4
                gwait(b)
                iwait_dst(j, b)
                scstart(b)

                @pl.when(jo < nb4 - 1)
                def _():
                    istart_src(j + 4, b)

                if b < 2:
                    @pl.when(jo > 0)
                    def _():
                        scwait(b2)
                    istart_dst(j + 2, b2)
                    iwait_src(j + 2, b2)
                    gstart(b2)
                else:
                    scwait(b2)

                    @pl.when(jo < nb4 - 1)
                    def _():
                        istart_dst(j + 2, b2)
                        iwait_src(j + 2, b2)
                        gstart(b2)
            return carry

        lax.fori_loop(0, nb4, body, 0)
        scwait(2)
        scwait(3)
        plsc.subcore_barrier()
        pltpu.sync_copy(acc.at[pl.ds(s * _RPT, _RPT)],
                        out.at[2 * p + c, pl.ds(s * _RPT, _RPT)])


_seg_call = pl.kernel(
    _seg_body,
    out_type=jax.ShapeDtypeStruct((_NG, _RA, _FW), jnp.float32),
    mesh=_mesh,
    compiler_params=pltpu.CompilerParams(use_tc_tiling_on_sc=False),
    scratch_types=(
        [pltpu.VMEM((_K,), jnp.int32) for _ in range(8)]
        + [pltpu.VMEM((_K, _FW), jnp.float32) for _ in range(4)]
        + [pltpu.VMEM((_RPT, _FW), jnp.float32),
           pltpu.VMEM_SHARED((_RA, _FW), jnp.float32)]
        + [pltpu.SemaphoreType.DMA for _ in range(16)]
    ),
)

def _deg_body(dst2, out, id0, id1, id2, id3, ones, zb, acc,
              sid0, sid1, sid2, sid3, ssc0, ssc1, ssc2, ssc3):
    c = lax.axis_index("c")
    s = lax.axis_index("s")
    ids = (id0, id1, id2, id3)
    sid = (sid0, sid1, sid2, sid3)
    ssc = (ssc0, ssc1, ssc2, ssc3)
    _fill(zb, _RPT, 0.0)
    _fill(ones, _K, 1.0)
    nb4 = _NBL // 8
    jbase = c * (_NBL // 2)

    def istart_dst(j, b):
        pltpu.make_async_copy(dst2.at[s, j], ids[b], sid[b]).start()

    def iwait_dst(j, b):
        pltpu.make_async_copy(dst2.at[s, j], ids[b], sid[b]).wait()

    def scstart(b):
        pltpu.make_async_copy(ones, acc.at[ids[b]], ssc[b]).start(add=True)

    def scwait(b):
        pltpu.make_async_copy(ones, acc.at[ids[b]], ssc[b]).wait()

    pltpu.sync_copy(zb, acc.at[pl.ds(s * _RPT, _RPT)])
    plsc.subcore_barrier()
    for b in range(2):
        istart_dst(jbase + b, b)

    def body(jo, carry):
        for b in range(4):
            j = jbase + 4 * jo + b
            b2 = (b + 2) % 4
            iwait_dst(j, b)
            scstart(b)
            if b < 2:
                @pl.when(jo > 0)
                def _():
                    scwait(b2)
                istart_dst(j + 2, b2)
            else:
                scwait(b2)

                @pl.when(jo < nb4 - 1)
                def _():
                    istart_dst(j + 2, b2)
        return carry

    lax.fori_loop(0, nb4, body, 0)
    scwait(2)
    scwait(3)
    plsc.subcore_barrier()
    pltpu.sync_copy(acc.at[pl.ds(s * _RPT, _RPT)],
                    out.at[c, pl.ds(s * _RPT, _RPT)])


_deg_call = pl.kernel(
    _deg_body,
    out_type=jax.ShapeDtypeStruct((_NC, _RA, _FW), jnp.float32),
    mesh=_mesh,
    compiler_params=pltpu.CompilerParams(use_tc_tiling_on_sc=False),
    scratch_types=(
        [pltpu.VMEM((_K,), jnp.int32) for _ in range(4)]
        + [pltpu.VMEM((_K, _FW), jnp.float32),
           pltpu.VMEM((_RPT, _FW), jnp.float32),
           pltpu.VMEM_SHARED((_RA, _FW), jnp.float32)]
        + [pltpu.SemaphoreType.DMA for _ in range(8)]
    ),
)


_RB = 1000
_GRID = _N // _RB


def _mm_t(a, w):
    return lax.dot_general(a, w, (((1,), (1,)), ((), ())),
                           preferred_element_type=jnp.float32)


def _split_groups(hw_ref, hw):
    for g in range(_NG):
        hw_ref[g] = hw[:, g * _FW:(g + 1) * _FW]


def _tc0_body(x_ref, win_ref, bin_ref, w1_ref, dd_ref, hw_ref, dis_ref):
    deg = dd_ref[0, :, 0:1] + dd_ref[1, :, 0:1] + 1.0
    dis = lax.rsqrt(deg)
    h0 = jnp.maximum(_mm_t(x_ref[...], win_ref[...]) + bin_ref[...], 0.0)
    hw = _mm_t(h0, w1_ref[...]) * dis
    _split_groups(hw_ref, hw)
    dis_ref[...] = dis


_tc0_call = pl.pallas_call(
    _tc0_body,
    grid=(_GRID,),
    in_specs=[
        pl.BlockSpec((_RB, _DIN), lambda i: (i, 0)),
        pl.BlockSpec((_DH, _DIN), lambda i: (0, 0)),
        pl.BlockSpec((1, _DH), lambda i: (0, 0)),
        pl.BlockSpec((_DH, _DH), lambda i: (0, 0)),
        pl.BlockSpec((_NC, _RB, _FW), lambda i: (0, i, 0)),
    ],
    out_specs=[
        pl.BlockSpec((_NG, _RB, _FW), lambda i: (0, i, 0)),
        pl.BlockSpec((_RB, 1), lambda i: (i, 0)),
    ],
    out_shape=[
        jax.ShapeDtypeStruct((_NG, _N, _FW), jnp.float32),
        jax.ShapeDtypeStruct((_N, 1), jnp.float32),
    ],
)


def _bn_relu(sd_ref, hwp_ref, dis_ref, b_ref, g_ref, be_ref, rm_ref, rv_ref):
    sc = g_ref[...] * lax.rsqrt(rv_ref[...] + 1e-5)
    tb = (b_ref[...] - rm_ref[...]) * sc + be_ref[...]
    seg = jnp.concatenate([sd_ref[g] + hwp_ref[g] for g in range(_NG)],
                          axis=1) * dis_ref[...]
    return jnp.maximum(seg * sc + tb, 0.0)


def _tcmid_body(sd_ref, hwp_ref, dis_ref, b_ref, g_ref, be_ref, rm_ref,
                rv_ref, wn_ref, hw_ref):
    h = _bn_relu(sd_ref, hwp_ref, dis_ref, b_ref, g_ref, be_ref, rm_ref, rv_ref)
    hw = _mm_t(h, wn_ref[...]) * dis_ref[...]
    _split_groups(hw_ref, hw)


_tcmid_call = pl.pallas_call(
    _tcmid_body,
    grid=(_GRID,),
    in_specs=[
        pl.BlockSpec((_NG, _RB, _FW), lambda i: (0, i, 0)),
        pl.BlockSpec((_NG, _RB, _FW), lambda i: (0, i, 0)),
        pl.BlockSpec((_RB, 1), lambda i: (i, 0)),
        pl.BlockSpec((1, _DH), lambda i: (0, 0)),
        pl.BlockSpec((1, _DH), lambda i: (0, 0)),
        pl.BlockSpec((1, _DH), lambda i: (0, 0)),
        pl.BlockSpec((1, _DH), lambda i: (0, 0)),
        pl.BlockSpec((1, _DH), lambda i: (0, 0)),
        pl.BlockSpec((_DH, _DH), lambda i: (0, 0)),
    ],
    out_specs=pl.BlockSpec((_NG, _RB, _FW), lambda i: (0, i, 0)),
    out_shape=jax.ShapeDtypeStruct((_NG, _N, _FW), jnp.float32),
)


def _tc3_body(sd_ref, hwp_ref, dis_ref, b_ref, g_ref, be_ref, rm_ref, rv_ref,
              batch_ref, wo1_ref, bo1_ref, wo2_ref, bo2_ref, out_ref,
              accp, accc):
    i = pl.program_id(0)

    @pl.when(i == 0)
    def _():
        accp[...] = jnp.zeros_like(accp)
        accc[...] = jnp.zeros_like(accc)

    h = _bn_relu(sd_ref, hwp_ref, dis_ref, b_ref, g_ref, be_ref, rm_ref, rv_ref)
    gid = lax.broadcasted_iota(jnp.int32, (_RB, _BG), 1)
    oh = (batch_ref[...] == gid).astype(jnp.float32)
    accp[...] += lax.dot_general(oh, h, (((0,), (0,)), ((), ())),
                                 preferred_element_type=jnp.float32)
    accc[...] += jnp.sum(oh, axis=0, keepdims=True)

    @pl.when(i == _GRID - 1)
    def _():
        cnt = jnp.reshape(jnp.maximum(accc[...], 1.0), (_BG, 1))
        pooled = accp[...] / cnt
        hid = jnp.maximum(_mm_t(pooled, wo1_ref[...]) + bo1_ref[...], 0.0)
        out_ref[...] = _mm_t(hid, wo2_ref[...]) + bo2_ref[...]


_tc3_call = pl.pallas_call(
    _tc3_body,
    grid=(_GRID,),
    in_specs=[
        pl.BlockSpec((_NG, _RB, _FW), lambda i: (0, i, 0)),
        pl.BlockSpec((_NG, _RB, _FW), lambda i: (0, i, 0)),
        pl.BlockSpec((_RB, 1), lambda i: (i, 0)),
        pl.BlockSpec((1, _DH), lambda i: (0, 0)),
        pl.BlockSpec((1, _DH), lambda i: (0, 0)),
        pl.BlockSpec((1, _DH), lambda i: (0, 0)),
        pl.BlockSpec((1, _DH), lambda i: (0, 0)),
        pl.BlockSpec((1, _DH), lambda i: (0, 0)),
        pl.BlockSpec((_RB, 1), lambda i: (i, 0)),
        pl.BlockSpec((_DH, _DH), lambda i: (0, 0)),
        pl.BlockSpec((1, _DH), lambda i: (0, 0)),
        pl.BlockSpec((_DE, _DH), lambda i: (0, 0)),
        pl.BlockSpec((1, _DE), lambda i: (0, 0)),
    ],
    out_specs=pl.BlockSpec((_BG, _DE), lambda i: (0, 0)),
    out_shape=jax.ShapeDtypeStruct((_BG, _DE), jnp.float32),
    scratch_shapes=[
        pltpu.VMEM((_BG, _DH), jnp.float32),
        pltpu.VMEM((1, _BG), jnp.float32),
    ],
)


def kernel(x, edge_index, batch, Win, bin_, W1, b1, g1, be1, rm1, rv1,
           W2, b2, g2, be2, rm2, rv2, W3, b3, g3, be3, rm3, rv3,
           Wo1, bo1, Wo2, bo2):
    src = edge_index[0]
    dst = edge_index[1]
    padlen = _EP - _E
    srcp = jnp.concatenate([src, jnp.zeros((padlen,), jnp.int32)])
    dstp = jnp.concatenate([dst, jnp.full((padlen,), _JR, jnp.int32)])
    src2 = jnp.stack([srcp + g * _N for g in range(_NG)]).reshape(
        _NP, _NC, _NS, _NBL, _K)
    dst2 = dstp.reshape(_NS, _NBL, _K)
    r = lambda v: v.reshape(1, -1)

    degdump = _deg_call(dst2)
    hw1, dis = _tc0_call(x, Win, r(bin_), W1, degdump)
    seg1 = _seg_call(hw1.reshape(_NG * _N, _FW), src2, dst2)
    hw2 = _tcmid_call(seg1, hw1, dis, r(b1), r(g1), r(be1), r(rm1), r(rv1), W2)
    seg2 = _seg_call(hw2.reshape(_NG * _N, _FW), src2, dst2)
    hw3 = _tcmid_call(seg2, hw2, dis, r(b2), r(g2), r(be2), r(rm2), r(rv2), W3)
    seg3 = _seg_call(hw3.reshape(_NG * _N, _FW), src2, dst2)
    out = _tc3_call(seg3, hw3, dis, r(b3), r(g3), r(be3), r(rm3), r(rv3),
                    batch.reshape(_N, 1), Wo1, r(bo1), Wo2, r(bo2))
    return out


# speedup vs baseline: 15.7693x; 1.0188x over previous
import jax
import jax.numpy as jnp
from jax import lax
from jax.experimental import pallas as pl
from jax.experimental.pallas import tpu as pltpu
from jax.experimental.pallas import tpu_sc as plsc

_N = 50000
_E = 800000
_DIN = 128
_DH = 64
_DE = 32
_BG = 64

_NC, _NS = 2, 16
_K = 128
_NBL = 392
_EP = _NS * _NBL * _K
_JR = _N
_FW = 16
_NG = 4
_NP = 2
_RA = 50048
_RPT = _RA // _NS

_mesh = plsc.VectorSubcoreMesh(core_axis_name="c", subcore_axis_name="s",
                               num_cores=_NC, num_subcores=_NS)


def _fill(ref, nrows, value):
    v = jnp.full((16,), value, jnp.float32)

    def body(r, carry):
        ref[r, pl.ds(0, 16)] = v
        return carry

    lax.fori_loop(0, nrows, body, 0)


def _seg_body(table, src2, dst2, out,
              is0, is1, is2, is3, id0, id1, id2, id3, r0, r1, r2, r3, zb, acc,
              sis0, sis1, sis2, sis3, sid0, sid1, sid2, sid3,
              sg0, sg1, sg2, sg3, ssc0, ssc1, ssc2, ssc3):
    c = lax.axis_index("c")
    s = lax.axis_index("s")
    iss = (is0, is1, is2, is3)
    ids = (id0, id1, id2, id3)
    rows = (r0, r1, r2, r3)
    sis = (sis0, sis1, sis2, sis3)
    sid = (sid0, sid1, sid2, sid3)
    sg = (sg0, sg1, sg2, sg3)
    ssc = (ssc0, ssc1, ssc2, ssc3)
    _fill(zb, _RPT, 0.0)
    nb4 = _NBL // 4

    def make(p):
        def istart_src(j, b):
            pltpu.make_async_copy(src2.at[p, c, s, j], iss[b], sis[b]).start()

        def iwait_src(j, b):
            pltpu.make_async_copy(src2.at[p, c, s, j], iss[b], sis[b]).wait()

        def istart_dst(j, b):
            pltpu.make_async_copy(dst2.at[s, j], ids[b], sid[b]).start()

        def iwait_dst(j, b):
            pltpu.make_async_copy(dst2.at[s, j], ids[b], sid[b]).wait()

        def gstart(b):
            pltpu.make_async_copy(table.at[iss[b]], rows[b], sg[b]).start()

        def gwait(b):
            pltpu.make_async_copy(table.at[iss[b]], rows[b], sg[b]).wait()

        def scstart(b):
            pltpu.make_async_copy(rows[b], acc.at[ids[b]], ssc[b]).start(add=True)

        def scwait(b):
            pltpu.make_async_copy(rows[b], acc.at[ids[b]], ssc[b]).wait()

        return (istart_src, iwait_src, istart_dst, iwait_dst, gstart, gwait,
                scstart, scwait)

    for p in range(_NP):
        (istart_src, iwait_src, istart_dst, iwait_dst, gstart, gwait,
         scstart, scwait) = make(p)
        pltpu.sync_copy(zb, acc.at[pl.ds(s * _RPT, _RPT)])
        plsc.subcore_barrier()

        for b in range(4):
            istart_src(b, b)
        for b in range(2):
            istart_dst(b, b)
        for b in range(2):
            iwait_src(b, b)
            gstart(b)

        def body(jo, carry):
            for b in range(4):
                j = 4 * jo + b
                b2 = (b + 2) % 4
                gwait(b)
                iwait_dst(j, b)
                scstart(b)

                @pl.when(jo < nb4 - 1)
                def _():
                    istart_src(j + 4, b)

                if b < 2:
                    @pl.when(jo > 0)
                    def _():
                        scwait(b2)
                    istart_dst(j + 2, b2)
                    iwait_src(j + 2, b2)
                    gstart(b2)
                else:
                    scwait(b2)

                    @pl.when(jo < nb4 - 1)
                    def _():
                        istart_dst(j + 2, b2)
                        iwait_src(j + 2, b2)
                        gstart(b2)
            return carry

        lax.fori_loop(0, nb4, body, 0)
        scwait(2)
        scwait(3)
        plsc.subcore_barrier()
        pltpu.sync_copy(acc.at[pl.ds(s * _RPT, _RPT)],
                        out.at[2 * p + c, pl.ds(s * _RPT, _RPT)])


_seg_call = pl.kernel(
    _seg_body,
    out_type=jax.ShapeDtypeStruct((_NG, _RA, _FW), jnp.float32),
    mesh=_mesh,
    compiler_params=pltpu.CompilerParams(use_tc_tiling_on_sc=False),
    scratch_types=(
        [pltpu.VMEM((_K,), jnp.int32) for _ in range(8)]
        + [pltpu.VMEM((_K, _FW), jnp.float32) for _ in range(4)]
        + [pltpu.VMEM((_RPT, _FW), jnp.float32),
           pltpu.VMEM_SHARED((_RA, _FW), jnp.float32)]
        + [pltpu.SemaphoreType.DMA for _ in range(16)]
    ),
)

def _deg_body(dst2, out, id0, id1, id2, id3, ones, zb, acc,
              sid0, sid1, sid2, sid3, ssc0, ssc1, ssc2, ssc3):
    c = lax.axis_index("c")
    s = lax.axis_index("s")
    ids = (id0, id1, id2, id3)
    sid = (sid0, sid1, sid2, sid3)
    ssc = (ssc0, ssc1, ssc2, ssc3)
    _fill(zb, _RPT, 0.0)
    _fill(ones, _K, 1.0)
    nb4 = _NBL // 8
    jbase = c * (_NBL // 2)

    def istart_dst(j, b):
        pltpu.make_async_copy(dst2.at[s, j], ids[b], sid[b]).start()

    def iwait_dst(j, b):
        pltpu.make_async_copy(dst2.at[s, j], ids[b], sid[b]).wait()

    def scstart(b):
        pltpu.make_async_copy(ones, acc.at[ids[b]], ssc[b]).start(add=True)

    def scwait(b):
        pltpu.make_async_copy(ones, acc.at[ids[b]], ssc[b]).wait()

    pltpu.sync_copy(zb, acc.at[pl.ds(s * _RPT, _RPT)])
    plsc.subcore_barrier()
    for b in range(2):
        istart_dst(jbase + b, b)

    def body(jo, carry):
        for b in range(4):
            j = jbase + 4 * jo + b
            b2 = (b + 2) % 4
            iwait_dst(j, b)
            scstart(b)
            if b < 2:
                @pl.when(jo > 0)
                def _():
                    scwait(b2)
                istart_dst(j + 2, b2)
            else:
                scwait(b2)

                @pl.when(jo < nb4 - 1)
                def _():
                    istart_dst(j + 2, b2)
        return carry

    lax.fori_loop(0, nb4, body, 0)
    scwait(2)
    scwait(3)
    plsc.subcore_barrier()
    pltpu.sync_copy(acc.at[pl.ds(s * _RPT, _RPT)],
                    out.at[c, pl.ds(s * _RPT, _RPT)])


_deg_call = pl.kernel(
    _deg_body,
    out_type=jax.ShapeDtypeStruct((_NC, _RA, _FW), jnp.float32),
    mesh=_mesh,
    compiler_params=pltpu.CompilerParams(use_tc_tiling_on_sc=False),
    scratch_types=(
        [pltpu.VMEM((_K,), jnp.int32) for _ in range(4)]
        + [pltpu.VMEM((_K, _FW), jnp.float32),
           pltpu.VMEM((_RPT, _FW), jnp.float32),
           pltpu.VMEM_SHARED((_RA, _FW), jnp.float32)]
        + [pltpu.SemaphoreType.DMA for _ in range(8)]
    ),
)


_RB = 2000
_GRID = _N // _RB


def _mm_t(a, w):
    return lax.dot_general(a, w, (((1,), (1,)), ((), ())),
                           preferred_element_type=jnp.float32)


def _split_groups(hw_ref, hw):
    for g in range(_NG):
        hw_ref[g] = hw[:, g * _FW:(g + 1) * _FW]


def _tc0_body(x_ref, win_ref, bin_ref, w1_ref, dd_ref, hw_ref, dis_ref):
    deg = dd_ref[0, :, 0:1] + dd_ref[1, :, 0:1] + 1.0
    dis = lax.rsqrt(deg)
    h0 = jnp.maximum(_mm_t(x_ref[...], win_ref[...]) + bin_ref[...], 0.0)
    hw = _mm_t(h0, w1_ref[...]) * dis
    _split_groups(hw_ref, hw)
    dis_ref[...] = dis


_tc0_call = pl.pallas_call(
    _tc0_body,
    grid=(_GRID,),
    in_specs=[
        pl.BlockSpec((_RB, _DIN), lambda i: (i, 0)),
        pl.BlockSpec((_DH, _DIN), lambda i: (0, 0)),
        pl.BlockSpec((1, _DH), lambda i: (0, 0)),
        pl.BlockSpec((_DH, _DH), lambda i: (0, 0)),
        pl.BlockSpec((_NC, _RB, _FW), lambda i: (0, i, 0)),
    ],
    out_specs=[
        pl.BlockSpec((_NG, _RB, _FW), lambda i: (0, i, 0)),
        pl.BlockSpec((_RB, 1), lambda i: (i, 0)),
    ],
    out_shape=[
        jax.ShapeDtypeStruct((_NG, _N, _FW), jnp.float32),
        jax.ShapeDtypeStruct((_N, 1), jnp.float32),
    ],
)


def _bn_relu(sd_ref, hwp_ref, dis_ref, b_ref, g_ref, be_ref, rm_ref, rv_ref):
    sc = g_ref[...] * lax.rsqrt(rv_ref[...] + 1e-5)
    tb = (b_ref[...] - rm_ref[...]) * sc + be_ref[...]
    seg = jnp.concatenate([sd_ref[g] + hwp_ref[g] for g in range(_NG)],
                          axis=1) * dis_ref[...]
    return jnp.maximum(seg * sc + tb, 0.0)


def _tcmid_body(sd_ref, hwp_ref, dis_ref, b_ref, g_ref, be_ref, rm_ref,
                rv_ref, wn_ref, hw_ref):
    h = _bn_relu(sd_ref, hwp_ref, dis_ref, b_ref, g_ref, be_ref, rm_ref, rv_ref)
    hw = _mm_t(h, wn_ref[...]) * dis_ref[...]
    _split_groups(hw_ref, hw)


_tcmid_call = pl.pallas_call(
    _tcmid_body,
    grid=(_GRID,),
    in_specs=[
        pl.BlockSpec((_NG, _RB, _FW), lambda i: (0, i, 0)),
        pl.BlockSpec((_NG, _RB, _FW), lambda i: (0, i, 0)),
        pl.BlockSpec((_RB, 1), lambda i: (i, 0)),
        pl.BlockSpec((1, _DH), lambda i: (0, 0)),
        pl.BlockSpec((1, _DH), lambda i: (0, 0)),
        pl.BlockSpec((1, _DH), lambda i: (0, 0)),
        pl.BlockSpec((1, _DH), lambda i: (0, 0)),
        pl.BlockSpec((1, _DH), lambda i: (0, 0)),
        pl.BlockSpec((_DH, _DH), lambda i: (0, 0)),
    ],
    out_specs=pl.BlockSpec((_NG, _RB, _FW), lambda i: (0, i, 0)),
    out_shape=jax.ShapeDtypeStruct((_NG, _N, _FW), jnp.float32),
)


def _tc3_body(sd_ref, hwp_ref, dis_ref, b_ref, g_ref, be_ref, rm_ref, rv_ref,
              batch_ref, wo1_ref, bo1_ref, wo2_ref, bo2_ref, out_ref,
              accp, accc):
    i = pl.program_id(0)

    @pl.when(i == 0)
    def _():
        accp[...] = jnp.zeros_like(accp)
        accc[...] = jnp.zeros_like(accc)

    h = _bn_relu(sd_ref, hwp_ref, dis_ref, b_ref, g_ref, be_ref, rm_ref, rv_ref)
    gid = lax.broadcasted_iota(jnp.int32, (_RB, _BG), 1)
    oh = (batch_ref[...] == gid).astype(jnp.float32)
    accp[...] += lax.dot_general(oh, h, (((0,), (0,)), ((), ())),
                                 preferred_element_type=jnp.float32)
    accc[...] += jnp.sum(oh, axis=0, keepdims=True)

    @pl.when(i == _GRID - 1)
    def _():
        cnt = jnp.reshape(jnp.maximum(accc[...], 1.0), (_BG, 1))
        pooled = accp[...] / cnt
        hid = jnp.maximum(_mm_t(pooled, wo1_ref[...]) + bo1_ref[...], 0.0)
        out_ref[...] = _mm_t(hid, wo2_ref[...]) + bo2_ref[...]


_tc3_call = pl.pallas_call(
    _tc3_body,
    grid=(_GRID,),
    in_specs=[
        pl.BlockSpec((_NG, _RB, _FW), lambda i: (0, i, 0)),
        pl.BlockSpec((_NG, _RB, _FW), lambda i: (0, i, 0)),
        pl.BlockSpec((_RB, 1), lambda i: (i, 0)),
        pl.BlockSpec((1, _DH), lambda i: (0, 0)),
        pl.BlockSpec((1, _DH), lambda i: (0, 0)),
        pl.BlockSpec((1, _DH), lambda i: (0, 0)),
        pl.BlockSpec((1, _DH), lambda i: (0, 0)),
        pl.BlockSpec((1, _DH), lambda i: (0, 0)),
        pl.BlockSpec((_RB, 1), lambda i: (i, 0)),
        pl.BlockSpec((_DH, _DH), lambda i: (0, 0)),
        pl.BlockSpec((1, _DH), lambda i: (0, 0)),
        pl.BlockSpec((_DE, _DH), lambda i: (0, 0)),
        pl.BlockSpec((1, _DE), lambda i: (0, 0)),
    ],
    out_specs=pl.BlockSpec((_BG, _DE), lambda i: (0, 0)),
    out_shape=jax.ShapeDtypeStruct((_BG, _DE), jnp.float32),
    scratch_shapes=[
        pltpu.VMEM((_BG, _DH), jnp.float32),
        pltpu.VMEM((1, _BG), jnp.float32),
    ],
)


def kernel(x, edge_index, batch, Win, bin_, W1, b1, g1, be1, rm1, rv1,
           W2, b2, g2, be2, rm2, rv2, W3, b3, g3, be3, rm3, rv3,
           Wo1, bo1, Wo2, bo2):
    src = edge_index[0]
    dst = edge_index[1]
    padlen = _EP - _E
    srcp = jnp.concatenate([src, jnp.zeros((padlen,), jnp.int32)])
    dstp = jnp.concatenate([dst, jnp.full((padlen,), _JR, jnp.int32)])
    src2 = jnp.stack([srcp + g * _N for g in range(_NG)]).reshape(
        _NP, _NC, _NS, _NBL, _K)
    dst2 = dstp.reshape(_NS, _NBL, _K)
    r = lambda v: v.reshape(1, -1)

    degdump = _deg_call(dst2)
    hw1, dis = _tc0_call(x, Win, r(bin_), W1, degdump)
    seg1 = _seg_call(hw1.reshape(_NG * _N, _FW), src2, dst2)
    hw2 = _tcmid_call(seg1, hw1, dis, r(b1), r(g1), r(be1), r(rm1), r(rv1), W2)
    seg2 = _seg_call(hw2.reshape(_NG * _N, _FW), src2, dst2)
    hw3 = _tcmid_call(seg2, hw2, dis, r(b2), r(g2), r(be2), r(rm2), r(rv2), W3)
    seg3 = _seg_call(hw3.reshape(_NG * _N, _FW), src2, dst2)
    out = _tc3_call(seg3, hw3, dis, r(b3), r(g3), r(be3), r(rm3), r(rv3),
                    batch.reshape(_N, 1), Wo1, r(bo1), Wo2, r(bo2))
    return out
